# baseline (speedup 1.0000x reference)
import jax
import jax.numpy as jnp
import numpy as np
from jax import lax
from jax.experimental import pallas as pl
from jax.experimental.pallas import tpu as pltpu
from jax.experimental.pallas import tpu_sc as plsc

L = 16
T = 2 ** 14
F = 2
N_MIN = 16.0
N_MAX = 512.0
B_GROWTH = float(np.exp((np.log(N_MAX) - np.log(N_MIN)) / (L - 1)))
NL = [float(np.floor(N_MIN * (B_GROWTH ** i))) for i in range(L)]

P1 = np.int32(np.uint32(2654435761).view(np.int32))
P2 = np.int32(805459861)
MASK = T - 1

N_POINTS = 262144
C = 4096
C16 = C // 16
NCORES = 2
NSUB = 16
NCH = (N_POINTS // NCORES) // C


def _body(x_hbm, tabs_hbm, out_hbm,
          x_v, tab_iv, tab0_v, tab1_v, fpair_v, gath_v, out_v, spbuf):
    cid = lax.axis_index("c")
    sid = lax.axis_index("s")
    half_base = cid * (N_POINTS // NCORES)

    iota = lax.iota(jnp.int32, 16)
    iota2 = iota * 2
    rbase = lax.shift_right_logical(iota, 1)
    cbase = jnp.bitwise_and(iota, 1)
    iota3 = iota * 3
    obase = rbase * 32 + cbase

    pltpu.sync_copy(tabs_hbm.at[sid], tab_iv)

    @plsc.parallel_loop(0, T // 16, unroll=4)
    def _deint(t):
        rows2 = iota2 + t * 32
        f0 = plsc.load_gather(tab_iv, [rows2])
        f1 = plsc.load_gather(tab_iv, [rows2 + 1])
        tab0_v[pl.ds(t * 16, 16)] = f0
        tab1_v[pl.ds(t * 16, 16)] = f1

    nl_vec = jnp.float32(NL[0])
    for _i in range(1, L):
        nl_vec = jnp.where(sid == _i, jnp.float32(NL[_i]), nl_vec)

    @pl.loop(0, NCH)
    def _chunk(k):
        row0 = half_base + k * C
        pltpu.sync_copy(x_hbm.at[pl.ds(row0 * 3, C * 3)], x_v)

        @plsc.parallel_loop(0, C // 16, unroll=4)
        def _grp(g):
            fx = iota3 + g * 48
            px = plsc.load_gather(x_v, [fx])
            py = plsc.load_gather(x_v, [fx + 1])
            pz = plsc.load_gather(x_v, [fx + 2])

            tx = px * nl_vec
            ty = py * nl_vec
            tz = pz * nl_vec
            gx = tx.astype(jnp.int32)
            gy = ty.astype(jnp.int32)
            gz = tz.astype(jnp.int32)
            wx = tx - gx.astype(jnp.float32)
            wy = ty - gy.astype(jnp.float32)
            wz = tz - gz.astype(jnp.float32)

            hy0 = gy * P1
            hy1 = hy0 + P1
            hz0 = gz * P2
            hz1 = hz0 + P2
            a0 = gx & MASK
            a1 = (gx + 1) & MASK
            b = [(hy0 ^ hz0) & MASK, (hy0 ^ hz1) & MASK,
                 (hy1 ^ hz0) & MASK, (hy1 ^ hz1) & MASK]

            res = []
            for tab in (tab0_v, tab1_v):
                yvals = []
                for jj in (0, 1):
                    zvals = []
                    for kk in (0, 1):
                        f0 = plsc.load_gather(tab, [a0 ^ b[2 * jj + kk]])
                        f1 = plsc.load_gather(tab, [a1 ^ b[2 * jj + kk]])
                        zvals.append(f0 + wx * (f1 - f0))
                    yvals.append(zvals[0] + wz * (zvals[1] - zvals[0]))
                res.append(yvals[0] + wy * (yvals[1] - yvals[0]))

            si = iota2 + g * 32
            plsc.store_scatter(fpair_v, [si], res[0])
            plsc.store_scatter(fpair_v, [si + 1], res[1])

        pltpu.sync_copy(fpair_v, spbuf.at[sid])
        plsc.subcore_barrier()
        @pl.loop(0, L)
        def _pull(lv):
            pltpu.sync_copy(spbuf.at[lv, pl.ds(sid * (C16 * 2), C16 * 2)],
                            gath_v.at[lv])
        plsc.subcore_barrier()

        @pl.loop(0, L)
        def _ilv(lv):
            t = obase + lv * 2
            trow = lax.shift_right_logical(t, 7)
            tcol = jnp.bitwise_and(t, 127)
            @pl.loop(0, (C16 * 2) // 16)
            def _blk(g):
                v = gath_v[lv, pl.ds(g * 16, 16)]
                plsc.store_scatter(out_v, [trow + g * 2, tcol], v)

        orow = pl.multiple_of((row0 + sid * C16) // 4, 8)
        pltpu.sync_copy(out_v, out_hbm.at[pl.ds(orow, C16 // 4), :])


@jax.jit
def kernel(x, tables):
    n = x.shape[0]
    mesh = plsc.VectorSubcoreMesh(core_axis_name="c", subcore_axis_name="s",
                                  num_cores=NCORES, num_subcores=NSUB)
    run = pl.kernel(
        _body,
        out_type=jax.ShapeDtypeStruct((n // 4, 128), jnp.float32),
        mesh=mesh,
        compiler_params=pltpu.CompilerParams(needs_layout_passes=False),
        scratch_types=[
            pltpu.VMEM((3 * C,), jnp.float32),
            pltpu.VMEM((2 * T,), jnp.float32),
            pltpu.VMEM((T,), jnp.float32),
            pltpu.VMEM((T,), jnp.float32),
            pltpu.VMEM((2 * C,), jnp.float32),
            pltpu.VMEM((L, C16 * 2), jnp.float32),
            pltpu.VMEM((C16 // 4, 128), jnp.float32),
            pltpu.VMEM_SHARED((L, 2 * C), jnp.float32),
        ],
    )
    return run(x.reshape(-1), tables.reshape(L, T * F)).reshape(n, L * F)


# speedup vs baseline: 1.1910x; 1.1910x over previous
import jax
import jax.numpy as jnp
import numpy as np
from jax import lax
from jax.experimental import pallas as pl
from jax.experimental.pallas import tpu as pltpu
from jax.experimental.pallas import tpu_sc as plsc

L = 16
T = 2 ** 14
F = 2
N_MIN = 16.0
N_MAX = 512.0
B_GROWTH = float(np.exp((np.log(N_MAX) - np.log(N_MIN)) / (L - 1)))
NL = [float(np.floor(N_MIN * (B_GROWTH ** i))) for i in range(L)]

P1 = np.int32(np.uint32(2654435761).view(np.int32))
P2 = np.int32(805459861)
MASK = T - 1

N_POINTS = 262144
C = 4096
C16 = C // 16
NCORES = 2
NSUB = 16
NCH = (N_POINTS // NCORES) // C


def _body(x_hbm, tab0_hbm, tab1_hbm, out_hbm,
          x_v, tab0_v, tab1_v, fpair_v, gath_v, out_v, spbuf, xsem, psem):
    cid = lax.axis_index("c")
    sid = lax.axis_index("s")
    half_base = cid * (N_POINTS // NCORES)

    pltpu.sync_copy(tab0_hbm.at[sid], tab0_v)
    pltpu.sync_copy(tab1_hbm.at[sid], tab1_v)

    nl_vec = jnp.float32(NL[0])
    for _i in range(1, L):
        nl_vec = jnp.where(sid == _i, jnp.float32(NL[_i]), nl_vec)

    iota = lax.iota(jnp.int32, 16)
    iota2 = iota * 2
    iota3 = iota * 3
    rbase = lax.shift_right_logical(iota, 1)
    cbase = jnp.bitwise_and(iota, 1)

    if True:
        pltpu.async_copy(x_hbm.at[pl.ds(half_base * 3, C * 3)],
                         x_v.at[pl.ds(0, C * 3)], xsem).wait()
        pltpu.async_copy(x_hbm.at[pl.ds(half_base * 3 + C * 3, C * 3)],
                         x_v.at[pl.ds(C * 3, C * 3)], xsem)

        @pl.loop(0, NCH)
        def _chunk(k):
            row0 = half_base + k * C
            buf = lax.rem(k, 2)
            xoff = iota3 + buf * (C * 3)

            @plsc.parallel_loop(0, C // 16, unroll=8)
            def _grp(g):
                fx = xoff + g * 48
                px = plsc.load_gather(x_v, [fx])
                py = plsc.load_gather(x_v, [fx + 1])
                pz = plsc.load_gather(x_v, [fx + 2])

                tx = px * nl_vec
                ty = py * nl_vec
                tz = pz * nl_vec
                gx = tx.astype(jnp.int32)
                gy = ty.astype(jnp.int32)
                gz = tz.astype(jnp.int32)
                wx = tx - gx.astype(jnp.float32)
                wy = ty - gy.astype(jnp.float32)
                wz = tz - gz.astype(jnp.float32)

                hy0 = gy * P1
                hy1 = hy0 + P1
                hz0 = gz * P2
                hz1 = hz0 + P2
                a0 = gx & MASK
                a1 = (gx + 1) & MASK
                b = [(hy0 ^ hz0) & MASK, (hy0 ^ hz1) & MASK,
                     (hy1 ^ hz0) & MASK, (hy1 ^ hz1) & MASK]

                res = []
                for tab in (tab0_v, tab1_v):
                    yvals = []
                    for jj in (0, 1):
                        zvals = []
                        for kk in (0, 1):
                            f0 = plsc.load_gather(tab, [a0 ^ b[2 * jj + kk]])
                            f1 = plsc.load_gather(tab, [a1 ^ b[2 * jj + kk]])
                            zvals.append(f0 + wx * (f1 - f0))
                        yvals.append(zvals[0] + wz * (zvals[1] - zvals[0]))
                    res.append(yvals[0] + wy * (yvals[1] - yvals[0]))

                si = iota2 + g * 32
                plsc.store_scatter(fpair_v, [si], res[0])
                plsc.store_scatter(fpair_v, [si + 1], res[1])

            @pl.when(k < NCH - 1)
            def _():
                pltpu.make_async_copy(
                    x_hbm.at[pl.ds((row0 + C) * 3, C * 3)],
                    x_v.at[pl.ds((1 - buf) * (C * 3), C * 3)], xsem).wait()
            @pl.when(k < NCH - 2)
            def _():
                pltpu.async_copy(
                    x_hbm.at[pl.ds((row0 + 2 * C) * 3, C * 3)],
                    x_v.at[pl.ds(buf * (C * 3), C * 3)], xsem)

            pltpu.sync_copy(fpair_v, spbuf.at[sid])
            plsc.subcore_barrier()
            copies = []
            for lv in range(L):
                copies.append(pltpu.async_copy(
                    spbuf.at[lv, pl.ds(sid * (C16 * 2), C16 * 2)],
                    gath_v.at[lv], psem))
            for cpy in copies:
                cpy.wait()
            plsc.subcore_barrier()

            @pl.loop(0, L)
            def _ilv(lv):
                cidx = cbase + lv * 2
                @plsc.parallel_loop(0, (C16 * 2) // 16, unroll=4)
                def _blk(g):
                    v = gath_v[lv, pl.ds(g * 16, 16)]
                    plsc.store_scatter(out_v, [rbase + g * 8, cidx], v)

            pltpu.sync_copy(out_v, out_hbm.at[pl.ds(row0 + sid * C16, C16), :])


@jax.jit
def kernel(x, tables):
    n = x.shape[0]
    mesh = plsc.VectorSubcoreMesh(core_axis_name="c", subcore_axis_name="s",
                                  num_cores=NCORES, num_subcores=NSUB)
    run = pl.kernel(
        _body,
        out_type=jax.ShapeDtypeStruct((n, L * F), jnp.float32),
        mesh=mesh,
        compiler_params=pltpu.CompilerParams(needs_layout_passes=False),
        scratch_types=[
            pltpu.VMEM((2 * 3 * C,), jnp.float32),
            pltpu.VMEM((T,), jnp.float32),
            pltpu.VMEM((T,), jnp.float32),
            pltpu.VMEM((2 * C,), jnp.float32),
            pltpu.VMEM((L, C16 * 2), jnp.float32),
            pltpu.VMEM((C16, L * F), jnp.float32),
            pltpu.VMEM_SHARED((L, 2 * C), jnp.float32),
            pltpu.SemaphoreType.DMA,
            pltpu.SemaphoreType.DMA,
        ],
    )
    return run(x.reshape(-1), tables[:, :, 0], tables[:, :, 1])


# speedup vs baseline: 1.3830x; 1.1613x over previous
import jax
import jax.numpy as jnp
import numpy as np
from jax import lax
from jax.experimental import pallas as pl
from jax.experimental.pallas import tpu as pltpu
from jax.experimental.pallas import tpu_sc as plsc

L = 16
T = 2 ** 14
F = 2
N_MIN = 16.0
N_MAX = 512.0
B_GROWTH = float(np.exp((np.log(N_MAX) - np.log(N_MIN)) / (L - 1)))
NL = [float(np.floor(N_MIN * (B_GROWTH ** i))) for i in range(L)]

P1 = np.int32(np.uint32(2654435761).view(np.int32))
P2 = np.int32(805459861)
MASK = T - 1

N_POINTS = 262144
C = 4096
C16 = C // 16
NCORES = 2
NSUB = 16
NCH = (N_POINTS // NCORES) // C


def _body(x_hbm, tab0_hbm, tab1_hbm, out_hbm,
          x_v, tab0_v, tab1_v, fpair_v, gath_v, out_v, spbuf, xsem, psem):
    cid = lax.axis_index("c")
    sid = lax.axis_index("s")
    half_base = cid * (N_POINTS // NCORES)

    pltpu.sync_copy(tab0_hbm.at[sid], tab0_v)
    pltpu.sync_copy(tab1_hbm.at[sid], tab1_v)

    nl_vec = jnp.float32(NL[0])
    for _i in range(1, L):
        nl_vec = jnp.where(sid == _i, jnp.float32(NL[_i]), nl_vec)

    iota = lax.iota(jnp.int32, 16)
    iota2 = iota * 2
    iota3 = iota * 3
    cbase = jnp.bitwise_and(iota, 1)
    lvlo = lax.shift_right_logical(iota, 1)
    lvhi = lvlo + 8

    if True:
        pltpu.async_copy(x_hbm.at[pl.ds(half_base * 3, C * 3)],
                         x_v.at[pl.ds(0, C * 3)], xsem).wait()
        pltpu.async_copy(x_hbm.at[pl.ds(half_base * 3 + C * 3, C * 3)],
                         x_v.at[pl.ds(C * 3, C * 3)], xsem)

        @pl.loop(0, NCH)
        def _chunk(k):
            row0 = half_base + k * C
            buf = lax.rem(k, 2)
            xoff = iota3 + buf * (C * 3)

            @plsc.parallel_loop(0, C // 16, unroll=4)
            def _grp(g):
                fx = xoff + g * 48
                px = plsc.load_gather(x_v, [fx])
                py = plsc.load_gather(x_v, [fx + 1])
                pz = plsc.load_gather(x_v, [fx + 2])

                tx = px * nl_vec
                ty = py * nl_vec
                tz = pz * nl_vec
                gx = tx.astype(jnp.int32)
                gy = ty.astype(jnp.int32)
                gz = tz.astype(jnp.int32)
                wx = tx - gx.astype(jnp.float32)
                wy = ty - gy.astype(jnp.float32)
                wz = tz - gz.astype(jnp.float32)

                hy0 = gy * P1
                hy1 = hy0 + P1
                hz0 = gz * P2
                hz1 = hz0 + P2
                a0 = gx & MASK
                a1 = (gx + 1) & MASK
                b = [(hy0 ^ hz0) & MASK, (hy0 ^ hz1) & MASK,
                     (hy1 ^ hz0) & MASK, (hy1 ^ hz1) & MASK]

                res = []
                for tab in (tab0_v, tab1_v):
                    yvals = []
                    for jj in (0, 1):
                        zvals = []
                        for kk in (0, 1):
                            f0 = plsc.load_gather(tab, [a0 ^ b[2 * jj + kk]])
                            f1 = plsc.load_gather(tab, [a1 ^ b[2 * jj + kk]])
                            zvals.append(f0 + wx * (f1 - f0))
                        yvals.append(zvals[0] + wz * (zvals[1] - zvals[0]))
                    res.append(yvals[0] + wy * (yvals[1] - yvals[0]))

                si = iota2 + g * 32
                plsc.store_scatter(fpair_v, [si], res[0])
                plsc.store_scatter(fpair_v, [si + 1], res[1])

            @pl.when(k < NCH - 1)
            def _():
                pltpu.make_async_copy(
                    x_hbm.at[pl.ds((row0 + C) * 3, C * 3)],
                    x_v.at[pl.ds((1 - buf) * (C * 3), C * 3)], xsem).wait()
            @pl.when(k < NCH - 2)
            def _():
                pltpu.async_copy(
                    x_hbm.at[pl.ds((row0 + 2 * C) * 3, C * 3)],
                    x_v.at[pl.ds(buf * (C * 3), C * 3)], xsem)

            pltpu.sync_copy(fpair_v, spbuf.at[sid])
            plsc.subcore_barrier()
            copies = []
            for lv in range(L):
                copies.append(pltpu.async_copy(
                    spbuf.at[lv, pl.ds(sid * (C16 * 2), C16 * 2)],
                    gath_v.at[lv], psem))
            for cpy in copies:
                cpy.wait()
            plsc.subcore_barrier()

            @plsc.parallel_loop(0, C16, unroll=4)
            def _row(r):
                i1 = cbase + r * 2
                v0 = plsc.load_gather(gath_v, [lvlo, i1])
                v1 = plsc.load_gather(gath_v, [lvhi, i1])
                out_v[r, pl.ds(0, 16)] = v0
                out_v[r, pl.ds(16, 16)] = v1

            pltpu.sync_copy(out_v, out_hbm.at[pl.ds(row0 + sid * C16, C16), :])


@jax.jit
def kernel(x, tables):
    n = x.shape[0]
    mesh = plsc.VectorSubcoreMesh(core_axis_name="c", subcore_axis_name="s",
                                  num_cores=NCORES, num_subcores=NSUB)
    run = pl.kernel(
        _body,
        out_type=jax.ShapeDtypeStruct((n, L * F), jnp.float32),
        mesh=mesh,
        compiler_params=pltpu.CompilerParams(needs_layout_passes=False),
        scratch_types=[
            pltpu.VMEM((2 * 3 * C,), jnp.float32),
            pltpu.VMEM((T,), jnp.float32),
            pltpu.VMEM((T,), jnp.float32),
            pltpu.VMEM((2 * C,), jnp.float32),
            pltpu.VMEM((L, C16 * 2), jnp.float32),
            pltpu.VMEM((C16, L * F), jnp.float32),
            pltpu.VMEM_SHARED((L, 2 * C), jnp.float32),
            pltpu.SemaphoreType.DMA,
            pltpu.SemaphoreType.DMA,
        ],
    )
    return run(x.reshape(-1), tables[:, :, 0], tables[:, :, 1])


# speedup vs baseline: 1.9086x; 1.3800x over previous
import jax
import jax.numpy as jnp
import numpy as np
from jax import lax
from jax.experimental import pallas as pl
from jax.experimental.pallas import tpu as pltpu
from jax.experimental.pallas import tpu_sc as plsc

L = 16
T = 2 ** 14
F = 2
N_MIN = 16.0
N_MAX = 512.0
B_GROWTH = float(np.exp((np.log(N_MAX) - np.log(N_MIN)) / (L - 1)))
NL = [float(np.floor(N_MIN * (B_GROWTH ** i))) for i in range(L)]

P1 = np.int32(np.uint32(2654435761).view(np.int32))
P2 = np.int32(805459861)
MASK = T - 1

N_POINTS = 262144
C = 4096
C16 = C // 16
NCORES = 2
NSUB = 16
NCH = (N_POINTS // NCORES) // C


def _body(x0_hbm, x1_hbm, x2_hbm, tab0_hbm, tab1_hbm, out_hbm,
          xc0_v, xc1_v, xc2_v, tab0_v, tab1_v, fpair_v, gath_v, out_v, spbuf,
          xsem, psem, osem):
    cid = lax.axis_index("c")
    sid = lax.axis_index("s")
    half_base = cid * (N_POINTS // NCORES)

    pltpu.sync_copy(tab0_hbm.at[sid], tab0_v)
    pltpu.sync_copy(tab1_hbm.at[sid], tab1_v)

    nl_vec = jnp.float32(NL[0])
    for _i in range(1, L):
        nl_vec = jnp.where(sid == _i, jnp.float32(NL[_i]), nl_vec)

    iota = lax.iota(jnp.int32, 16)
    iota2 = iota * 2
    cbase = jnp.bitwise_and(iota, 1)
    lvlo = lax.shift_right_logical(iota, 1)
    lvhi = lvlo + 8

    xcols = ((x0_hbm, xc0_v), (x1_hbm, xc1_v), (x2_hbm, xc2_v))
    for h, v in xcols:
        pltpu.async_copy(h.at[pl.ds(half_base, C)], v.at[pl.ds(0, C)],
                         xsem).wait()
    for h, v in xcols:
        pltpu.async_copy(h.at[pl.ds(half_base + C, C)], v.at[pl.ds(C, C)],
                         xsem)

    @pl.loop(0, NCH)
    def _chunk(k):
        row0 = half_base + k * C
        buf = lax.rem(k, 2)
        xb = buf * C

        @plsc.parallel_loop(0, C // 16, unroll=4)
        def _grp(g):
            base = xb + g * 16
            px = xc0_v[pl.ds(base, 16)]
            py = xc1_v[pl.ds(base, 16)]
            pz = xc2_v[pl.ds(base, 16)]

            tx = px * nl_vec
            ty = py * nl_vec
            tz = pz * nl_vec
            gx = tx.astype(jnp.int32)
            gy = ty.astype(jnp.int32)
            gz = tz.astype(jnp.int32)
            wx = tx - gx.astype(jnp.float32)
            wy = ty - gy.astype(jnp.float32)
            wz = tz - gz.astype(jnp.float32)

            hy0 = gy * P1
            hy1 = hy0 + P1
            hz0 = gz * P2
            hz1 = hz0 + P2
            a0 = gx & MASK
            a1 = (gx + 1) & MASK
            b = [(hy0 ^ hz0) & MASK, (hy0 ^ hz1) & MASK,
                 (hy1 ^ hz0) & MASK, (hy1 ^ hz1) & MASK]

            res = []
            for tab in (tab0_v, tab1_v):
                yvals = []
                for jj in (0, 1):
                    zvals = []
                    for kk in (0, 1):
                        f0 = plsc.load_gather(tab, [a0 ^ b[2 * jj + kk]])
                        f1 = plsc.load_gather(tab, [a1 ^ b[2 * jj + kk]])
                        zvals.append(f0 + wx * (f1 - f0))
                    yvals.append(zvals[0] + wz * (zvals[1] - zvals[0]))
                res.append(yvals[0] + wy * (yvals[1] - yvals[0]))

            si = iota2 + g * 32
            plsc.store_scatter(fpair_v, [si], res[0])
            plsc.store_scatter(fpair_v, [si + 1], res[1])

        @pl.when(k < NCH - 1)
        def _():
            for h, v in xcols:
                pltpu.make_async_copy(
                    h.at[pl.ds(row0 + C, C)],
                    v.at[pl.ds((1 - buf) * C, C)], xsem).wait()
        @pl.when(k < NCH - 2)
        def _():
            for h, v in xcols:
                pltpu.async_copy(h.at[pl.ds(row0 + 2 * C, C)],
                                 v.at[pl.ds(buf * C, C)], xsem)

        pltpu.sync_copy(fpair_v, spbuf.at[sid])
        plsc.subcore_barrier()
        copies = []
        for lv in range(L):
            copies.append(pltpu.async_copy(
                spbuf.at[lv, pl.ds(sid * (C16 * 2), C16 * 2)],
                gath_v.at[lv], psem))
        for cpy in copies:
            cpy.wait()
        plsc.subcore_barrier()

        @pl.when(k > 0)
        def _():
            pltpu.make_async_copy(
                out_v, out_hbm.at[pl.ds(row0 + sid * C16, C16), :],
                osem).wait()

        @plsc.parallel_loop(0, C16, unroll=4)
        def _row(r):
            i1 = cbase + r * 2
            v0 = plsc.load_gather(gath_v, [lvlo, i1])
            v1 = plsc.load_gather(gath_v, [lvhi, i1])
            out_v[r, pl.ds(0, 16)] = v0
            out_v[r, pl.ds(16, 16)] = v1

        pltpu.async_copy(out_v, out_hbm.at[pl.ds(row0 + sid * C16, C16), :],
                         osem)

    pltpu.make_async_copy(
        out_v, out_hbm.at[pl.ds(half_base + sid * C16, C16), :], osem).wait()


@jax.jit
def kernel(x, tables):
    n = x.shape[0]
    mesh = plsc.VectorSubcoreMesh(core_axis_name="c", subcore_axis_name="s",
                                  num_cores=NCORES, num_subcores=NSUB)
    run = pl.kernel(
        _body,
        out_type=jax.ShapeDtypeStruct((n, L * F), jnp.float32),
        mesh=mesh,
        compiler_params=pltpu.CompilerParams(needs_layout_passes=False),
        scratch_types=[
            pltpu.VMEM((2 * C,), jnp.float32),
            pltpu.VMEM((2 * C,), jnp.float32),
            pltpu.VMEM((2 * C,), jnp.float32),
            pltpu.VMEM((T,), jnp.float32),
            pltpu.VMEM((T,), jnp.float32),
            pltpu.VMEM((2 * C,), jnp.float32),
            pltpu.VMEM((L, C16 * 2), jnp.float32),
            pltpu.VMEM((C16, L * F), jnp.float32),
            pltpu.VMEM_SHARED((L, 2 * C), jnp.float32),
            pltpu.SemaphoreType.DMA,
            pltpu.SemaphoreType.DMA,
            pltpu.SemaphoreType.DMA,
        ],
    )
    return run(x[:, 0], x[:, 1], x[:, 2], tables[:, :, 0], tables[:, :, 1])


# speedup vs baseline: 2.0739x; 1.0866x over previous
import jax
import jax.numpy as jnp
import numpy as np
from jax import lax
from jax.experimental import pallas as pl
from jax.experimental.pallas import tpu as pltpu
from jax.experimental.pallas import tpu_sc as plsc

L = 16
T = 2 ** 14
F = 2
N_MIN = 16.0
N_MAX = 512.0
B_GROWTH = float(np.exp((np.log(N_MAX) - np.log(N_MIN)) / (L - 1)))
NL = [float(np.floor(N_MIN * (B_GROWTH ** i))) for i in range(L)]

P1 = np.int32(np.uint32(2654435761).view(np.int32))
P2 = np.int32(805459861)
MASK = T - 1

N_POINTS = 262144
C = 4096
C16 = C // 16
NCORES = 2
NSUB = 16
NCH = (N_POINTS // NCORES) // C


def _body(x0_hbm, x1_hbm, x2_hbm, tab0_hbm, tab1_hbm, out_hbm,
          xc0_v, xc1_v, xc2_v, tab0_v, tab1_v, fpair_v, gath_v, out_v, spbuf,
          xsem, psem, osem):
    cid = lax.axis_index("c")
    sid = lax.axis_index("s")
    half_base = cid * (N_POINTS // NCORES)

    pltpu.sync_copy(tab0_hbm.at[sid], tab0_v)
    pltpu.sync_copy(tab1_hbm.at[sid], tab1_v)

    nl_vec = jnp.float32(NL[0])
    for _i in range(1, L):
        nl_vec = jnp.where(sid == _i, jnp.float32(NL[_i]), nl_vec)

    iota = lax.iota(jnp.int32, 16)
    iota2 = iota * 2
    cbase = jnp.bitwise_and(iota, 1)
    lvlo = lax.shift_right_logical(iota, 1)
    lvhi = lvlo + 8

    xcols = ((x0_hbm, xc0_v), (x1_hbm, xc1_v), (x2_hbm, xc2_v))
    for h, v in xcols:
        pltpu.async_copy(h.at[pl.ds(half_base, C)], v.at[pl.ds(0, C)],
                         xsem).wait()
    for h, v in xcols:
        pltpu.async_copy(h.at[pl.ds(half_base + C, C)], v.at[pl.ds(C, C)],
                         xsem)

    @pl.loop(0, NCH)
    def _chunk(k):
        row0 = half_base + k * C
        buf = lax.rem(k, 2)
        xb = buf * C

        @plsc.parallel_loop(0, C // 16, unroll=4)
        def _grp(g):
            base = xb + g * 16
            px = xc0_v[pl.ds(base, 16)]
            py = xc1_v[pl.ds(base, 16)]
            pz = xc2_v[pl.ds(base, 16)]

            tx = px * nl_vec
            ty = py * nl_vec
            tz = pz * nl_vec
            gx = tx.astype(jnp.int32)
            gy = ty.astype(jnp.int32)
            gz = tz.astype(jnp.int32)
            wx = tx - gx.astype(jnp.float32)
            wy = ty - gy.astype(jnp.float32)
            wz = tz - gz.astype(jnp.float32)

            hy0 = gy * P1
            hy1 = hy0 + P1
            hz0 = gz * P2
            hz1 = hz0 + P2
            a0 = gx
            a1 = gx + 1
            b = [(hy0 ^ hz0) & MASK, (hy0 ^ hz1) & MASK,
                 (hy1 ^ hz0) & MASK, (hy1 ^ hz1) & MASK]

            res = []
            for tab in (tab0_v, tab1_v):
                yvals = []
                for jj in (0, 1):
                    zvals = []
                    for kk in (0, 1):
                        f0 = plsc.load_gather(tab, [a0 ^ b[2 * jj + kk]])
                        f1 = plsc.load_gather(tab, [a1 ^ b[2 * jj + kk]])
                        zvals.append(f0 + wx * (f1 - f0))
                    yvals.append(zvals[0] + wz * (zvals[1] - zvals[0]))
                res.append(yvals[0] + wy * (yvals[1] - yvals[0]))

            si = iota2 + g * 32
            plsc.store_scatter(fpair_v, [si], res[0])
            plsc.store_scatter(fpair_v, [si + 1], res[1])

        @pl.when(k < NCH - 1)
        def _():
            for h, v in xcols:
                pltpu.make_async_copy(
                    h.at[pl.ds(row0 + C, C)],
                    v.at[pl.ds((1 - buf) * C, C)], xsem).wait()
        @pl.when(k < NCH - 2)
        def _():
            for h, v in xcols:
                pltpu.async_copy(h.at[pl.ds(row0 + 2 * C, C)],
                                 v.at[pl.ds(buf * C, C)], xsem)

        pltpu.sync_copy(fpair_v, spbuf.at[sid])
        plsc.subcore_barrier()
        pltpu.sync_copy(spbuf.at[:, pl.ds(sid * (C16 * 2), C16 * 2)], gath_v)
        plsc.subcore_barrier()

        @pl.when(k > 0)
        def _():
            pltpu.make_async_copy(
                out_v, out_hbm.at[pl.ds(row0 + sid * C16, C16), :],
                osem).wait()

        @plsc.parallel_loop(0, C16, unroll=4)
        def _row(r):
            i1 = cbase + r * 2
            v0 = plsc.load_gather(gath_v, [lvlo, i1])
            v1 = plsc.load_gather(gath_v, [lvhi, i1])
            out_v[r, pl.ds(0, 16)] = v0
            out_v[r, pl.ds(16, 16)] = v1

        pltpu.async_copy(out_v, out_hbm.at[pl.ds(row0 + sid * C16, C16), :],
                         osem)

    pltpu.make_async_copy(
        out_v, out_hbm.at[pl.ds(half_base + sid * C16, C16), :], osem).wait()


@jax.jit
def kernel(x, tables):
    n = x.shape[0]
    mesh = plsc.VectorSubcoreMesh(core_axis_name="c", subcore_axis_name="s",
                                  num_cores=NCORES, num_subcores=NSUB)
    run = pl.kernel(
        _body,
        out_type=jax.ShapeDtypeStruct((n, L * F), jnp.float32),
        mesh=mesh,
        compiler_params=pltpu.CompilerParams(needs_layout_passes=False),
        scratch_types=[
            pltpu.VMEM((2 * C,), jnp.float32),
            pltpu.VMEM((2 * C,), jnp.float32),
            pltpu.VMEM((2 * C,), jnp.float32),
            pltpu.VMEM((T,), jnp.float32),
            pltpu.VMEM((T,), jnp.float32),
            pltpu.VMEM((2 * C,), jnp.float32),
            pltpu.VMEM((L, C16 * 2), jnp.float32),
            pltpu.VMEM((C16, L * F), jnp.float32),
            pltpu.VMEM_SHARED((L, 2 * C), jnp.float32),
            pltpu.SemaphoreType.DMA,
            pltpu.SemaphoreType.DMA,
            pltpu.SemaphoreType.DMA,
        ],
    )
    return run(x[:, 0], x[:, 1], x[:, 2], tables[:, :, 0], tables[:, :, 1])


# speedup vs baseline: 2.0931x; 1.0092x over previous
import jax
import jax.numpy as jnp
import numpy as np
from jax import lax
from jax.experimental import pallas as pl
from jax.experimental.pallas import tpu as pltpu
from jax.experimental.pallas import tpu_sc as plsc

L = 16
T = 2 ** 14
F = 2
N_MIN = 16.0
N_MAX = 512.0
B_GROWTH = float(np.exp((np.log(N_MAX) - np.log(N_MIN)) / (L - 1)))
NL = [float(np.floor(N_MIN * (B_GROWTH ** i))) for i in range(L)]

P1 = np.int32(np.uint32(2654435761).view(np.int32))
P2 = np.int32(805459861)
MASK = T - 1

N_POINTS = 262144
C = 4096
C16 = C // 16
NCORES = 2
NSUB = 16
NCH = (N_POINTS // NCORES) // C


def _body(x0_hbm, x1_hbm, x2_hbm, tab0_hbm, tab1_hbm, out_hbm,
          xc0_v, xc1_v, xc2_v, tab0_v, tab1_v, fpair_v, gath_v, out_v, spbuf,
          xsem, psem, osem):
    cid = lax.axis_index("c")
    sid = lax.axis_index("s")
    half_base = cid * (N_POINTS // NCORES)

    pltpu.sync_copy(tab0_hbm.at[sid], tab0_v)
    pltpu.sync_copy(tab1_hbm.at[sid], tab1_v)

    nl_vec = jnp.float32(NL[0])
    for _i in range(1, L):
        nl_vec = jnp.where(sid == _i, jnp.float32(NL[_i]), nl_vec)

    iota = lax.iota(jnp.int32, 16)
    iota2 = iota * 2
    cbase = jnp.bitwise_and(iota, 1)
    lvlo = lax.shift_right_logical(iota, 1)
    lvhi = lvlo + 8

    xcols = ((x0_hbm, xc0_v), (x1_hbm, xc1_v), (x2_hbm, xc2_v))
    for h, v in xcols:
        pltpu.async_copy(h.at[pl.ds(half_base, C)], v.at[pl.ds(0, C)],
                         xsem).wait()
    for h, v in xcols:
        pltpu.async_copy(h.at[pl.ds(half_base + C, C)], v.at[pl.ds(C, C)],
                         xsem)

    @pl.loop(0, NCH)
    def _chunk(k):
        row0 = half_base + k * C
        buf = lax.rem(k, 2)
        xb = buf * C

        @plsc.parallel_loop(0, C // 16, unroll=6)
        def _grp(g):
            base = xb + g * 16
            px = xc0_v[pl.ds(base, 16)]
            py = xc1_v[pl.ds(base, 16)]
            pz = xc2_v[pl.ds(base, 16)]

            tx = px * nl_vec
            ty = py * nl_vec
            tz = pz * nl_vec
            gx = tx.astype(jnp.int32)
            gy = ty.astype(jnp.int32)
            gz = tz.astype(jnp.int32)
            wx = tx - gx.astype(jnp.float32)
            wy = ty - gy.astype(jnp.float32)
            wz = tz - gz.astype(jnp.float32)

            hy0 = gy * P1
            hy1 = hy0 + P1
            hz0 = gz * P2
            hz1 = hz0 + P2
            a0 = gx
            a1 = gx + 1
            b = [(hy0 ^ hz0) & MASK, (hy0 ^ hz1) & MASK,
                 (hy1 ^ hz0) & MASK, (hy1 ^ hz1) & MASK]

            res = []
            for tab in (tab0_v, tab1_v):
                yvals = []
                for jj in (0, 1):
                    zvals = []
                    for kk in (0, 1):
                        f0 = plsc.load_gather(tab, [a0 ^ b[2 * jj + kk]])
                        f1 = plsc.load_gather(tab, [a1 ^ b[2 * jj + kk]])
                        zvals.append(f0 + wx * (f1 - f0))
                    yvals.append(zvals[0] + wz * (zvals[1] - zvals[0]))
                res.append(yvals[0] + wy * (yvals[1] - yvals[0]))

            si = iota2 + g * 32
            plsc.store_scatter(fpair_v, [si], res[0])
            plsc.store_scatter(fpair_v, [si + 1], res[1])

        @pl.when(k < NCH - 1)
        def _():
            for h, v in xcols:
                pltpu.make_async_copy(
                    h.at[pl.ds(row0 + C, C)],
                    v.at[pl.ds((1 - buf) * C, C)], xsem).wait()
        @pl.when(k < NCH - 2)
        def _():
            for h, v in xcols:
                pltpu.async_copy(h.at[pl.ds(row0 + 2 * C, C)],
                                 v.at[pl.ds(buf * C, C)], xsem)

        pltpu.sync_copy(fpair_v, spbuf.at[sid])
        plsc.subcore_barrier()
        pltpu.sync_copy(spbuf.at[:, pl.ds(sid * (C16 * 2), C16 * 2)], gath_v)
        plsc.subcore_barrier()

        @pl.when(k > 0)
        def _():
            pltpu.make_async_copy(
                out_v, out_hbm.at[pl.ds(row0 + sid * C16, C16), :],
                osem).wait()

        @plsc.parallel_loop(0, C16, unroll=4)
        def _row(r):
            i1 = cbase + r * 2
            v0 = plsc.load_gather(gath_v, [lvlo, i1])
            v1 = plsc.load_gather(gath_v, [lvhi, i1])
            out_v[r, pl.ds(0, 16)] = v0
            out_v[r, pl.ds(16, 16)] = v1

        pltpu.async_copy(out_v, out_hbm.at[pl.ds(row0 + sid * C16, C16), :],
                         osem)

    pltpu.make_async_copy(
        out_v, out_hbm.at[pl.ds(half_base + sid * C16, C16), :], osem).wait()


@jax.jit
def kernel(x, tables):
    n = x.shape[0]
    mesh = plsc.VectorSubcoreMesh(core_axis_name="c", subcore_axis_name="s",
                                  num_cores=NCORES, num_subcores=NSUB)
    run = pl.kernel(
        _body,
        out_type=jax.ShapeDtypeStruct((n, L * F), jnp.float32),
        mesh=mesh,
        compiler_params=pltpu.CompilerParams(needs_layout_passes=False),
        scratch_types=[
            pltpu.VMEM((2 * C,), jnp.float32),
            pltpu.VMEM((2 * C,), jnp.float32),
            pltpu.VMEM((2 * C,), jnp.float32),
            pltpu.VMEM((T,), jnp.float32),
            pltpu.VMEM((T,), jnp.float32),
            pltpu.VMEM((2 * C,), jnp.float32),
            pltpu.VMEM((L, C16 * 2), jnp.float32),
            pltpu.VMEM((C16, L * F), jnp.float32),
            pltpu.VMEM_SHARED((L, 2 * C), jnp.float32),
            pltpu.SemaphoreType.DMA,
            pltpu.SemaphoreType.DMA,
            pltpu.SemaphoreType.DMA,
        ],
    )
    return run(x[:, 0], x[:, 1], x[:, 2], tables[:, :, 0], tables[:, :, 1])


# speedup vs baseline: 2.1488x; 1.0266x over previous
import jax
import jax.numpy as jnp
import numpy as np
from jax import lax
from jax.experimental import pallas as pl
from jax.experimental.pallas import tpu as pltpu
from jax.experimental.pallas import tpu_sc as plsc

L = 16
T = 2 ** 14
F = 2
N_MIN = 16.0
N_MAX = 512.0
B_GROWTH = float(np.exp((np.log(N_MAX) - np.log(N_MIN)) / (L - 1)))
NL = [float(np.floor(N_MIN * (B_GROWTH ** i))) for i in range(L)]

P1 = np.int32(np.uint32(2654435761).view(np.int32))
P2 = np.int32(805459861)
MASK = T - 1

N_POINTS = 262144
C = 4096
C16 = C // 16
NCORES = 2
NSUB = 16
NCH = (N_POINTS // NCORES) // C


def _interleave(gath_v, out_v, cbase, lvlo, lvhi):
    @plsc.parallel_loop(0, C16, unroll=4)
    def _row(r):
        i1 = cbase + r * 2
        v0 = plsc.load_gather(gath_v, [lvlo, i1])
        v1 = plsc.load_gather(gath_v, [lvhi, i1])
        out_v[r, pl.ds(0, 16)] = v0
        out_v[r, pl.ds(16, 16)] = v1


def _body(x0_hbm, x1_hbm, x2_hbm, tab0_hbm, tab1_hbm, out_hbm,
          xc0_v, xc1_v, xc2_v, tab0_v, tab1_v, fpair_v, gath_v, out_v, spbuf,
          xsem, psem, osem):
    cid = lax.axis_index("c")
    sid = lax.axis_index("s")
    half_base = cid * (N_POINTS // NCORES)

    pltpu.sync_copy(tab0_hbm.at[sid], tab0_v)
    pltpu.sync_copy(tab1_hbm.at[sid], tab1_v)

    nl_vec = jnp.float32(NL[0])
    for _i in range(1, L):
        nl_vec = jnp.where(sid == _i, jnp.float32(NL[_i]), nl_vec)

    iota = lax.iota(jnp.int32, 16)
    iota2 = iota * 2
    cbase = jnp.bitwise_and(iota, 1)
    lvlo = lax.shift_right_logical(iota, 1)
    lvhi = lvlo + 8

    xcols = ((x0_hbm, xc0_v), (x1_hbm, xc1_v), (x2_hbm, xc2_v))
    for h, v in xcols:
        pltpu.async_copy(h.at[pl.ds(half_base, C)], v.at[pl.ds(0, C)],
                         xsem).wait()
    for h, v in xcols:
        pltpu.async_copy(h.at[pl.ds(half_base + C, C)], v.at[pl.ds(C, C)],
                         xsem)

    @pl.loop(0, NCH)
    def _chunk(k):
        row0 = half_base + k * C
        buf = lax.rem(k, 2)
        xb = buf * C

        @plsc.parallel_loop(0, C // 16, unroll=6)
        def _grp(g):
            base = xb + g * 16
            px = xc0_v[pl.ds(base, 16)]
            py = xc1_v[pl.ds(base, 16)]
            pz = xc2_v[pl.ds(base, 16)]

            tx = px * nl_vec
            ty = py * nl_vec
            tz = pz * nl_vec
            gx = tx.astype(jnp.int32)
            gy = ty.astype(jnp.int32)
            gz = tz.astype(jnp.int32)
            wx = tx - gx.astype(jnp.float32)
            wy = ty - gy.astype(jnp.float32)
            wz = tz - gz.astype(jnp.float32)

            hy0 = gy * P1
            hy1 = hy0 + P1
            hz0 = gz * P2
            hz1 = hz0 + P2
            a0 = gx
            a1 = gx + 1
            b = [(hy0 ^ hz0) & MASK, (hy0 ^ hz1) & MASK,
                 (hy1 ^ hz0) & MASK, (hy1 ^ hz1) & MASK]

            res = []
            for tab in (tab0_v, tab1_v):
                yvals = []
                for jj in (0, 1):
                    zvals = []
                    for kk in (0, 1):
                        f0 = plsc.load_gather(tab, [a0 ^ b[2 * jj + kk]])
                        f1 = plsc.load_gather(tab, [a1 ^ b[2 * jj + kk]])
                        zvals.append(f0 + wx * (f1 - f0))
                    yvals.append(zvals[0] + wz * (zvals[1] - zvals[0]))
                res.append(yvals[0] + wy * (yvals[1] - yvals[0]))

            si = iota2 + g * 32
            plsc.store_scatter(fpair_v, [si], res[0])
            plsc.store_scatter(fpair_v, [si + 1], res[1])

        @pl.when(k < NCH - 1)
        def _():
            for h, v in xcols:
                pltpu.make_async_copy(
                    h.at[pl.ds(row0 + C, C)],
                    v.at[pl.ds((1 - buf) * C, C)], xsem).wait()
        @pl.when(k < NCH - 2)
        def _():
            for h, v in xcols:
                pltpu.async_copy(h.at[pl.ds(row0 + 2 * C, C)],
                                 v.at[pl.ds(buf * C, C)], xsem)

        @pl.when(k > 0)
        def _():
            pltpu.make_async_copy(
                spbuf.at[:, pl.ds(sid * (C16 * 2), C16 * 2)], gath_v,
                psem).wait()
        plsc.subcore_barrier()
        @pl.when(k > 1)
        def _():
            pltpu.make_async_copy(
                out_v, out_hbm.at[pl.ds(row0 + sid * C16, C16), :],
                osem).wait()
        @pl.when(k > 0)
        def _():
            _interleave(gath_v, out_v, cbase, lvlo, lvhi)
            pltpu.async_copy(
                out_v,
                out_hbm.at[pl.ds(row0 - C + sid * C16, C16), :], osem)

        pltpu.sync_copy(fpair_v, spbuf.at[sid])
        plsc.subcore_barrier()
        pltpu.async_copy(spbuf.at[:, pl.ds(sid * (C16 * 2), C16 * 2)],
                         gath_v, psem)

    pltpu.make_async_copy(
        spbuf.at[:, pl.ds(sid * (C16 * 2), C16 * 2)], gath_v, psem).wait()
    pltpu.make_async_copy(
        out_v, out_hbm.at[pl.ds(half_base + sid * C16, C16), :], osem).wait()
    _interleave(gath_v, out_v, cbase, lvlo, lvhi)
    last = half_base + (NCH - 1) * C + sid * C16
    pltpu.async_copy(out_v, out_hbm.at[pl.ds(last, C16), :], osem)
    pltpu.make_async_copy(
        out_v, out_hbm.at[pl.ds(last, C16), :], osem).wait()


@jax.jit
def kernel(x, tables):
    n = x.shape[0]
    mesh = plsc.VectorSubcoreMesh(core_axis_name="c", subcore_axis_name="s",
                                  num_cores=NCORES, num_subcores=NSUB)
    run = pl.kernel(
        _body,
        out_type=jax.ShapeDtypeStruct((n, L * F), jnp.float32),
        mesh=mesh,
        compiler_params=pltpu.CompilerParams(needs_layout_passes=False),
        scratch_types=[
            pltpu.VMEM((2 * C,), jnp.float32),
            pltpu.VMEM((2 * C,), jnp.float32),
            pltpu.VMEM((2 * C,), jnp.float32),
            pltpu.VMEM((T,), jnp.float32),
            pltpu.VMEM((T,), jnp.float32),
            pltpu.VMEM((2 * C,), jnp.float32),
            pltpu.VMEM((L, C16 * 2), jnp.float32),
            pltpu.VMEM((C16, L * F), jnp.float32),
            pltpu.VMEM_SHARED((L, 2 * C), jnp.float32),
            pltpu.SemaphoreType.DMA,
            pltpu.SemaphoreType.DMA,
            pltpu.SemaphoreType.DMA,
        ],
    )
    return run(x[:, 0], x[:, 1], x[:, 2], tables[:, :, 0], tables[:, :, 1])
